# Initial kernel scaffold; baseline (speedup 1.0000x reference)
import jax
import jax.numpy as jnp
from jax.experimental import pallas as pl


def kernel(img, params):
    raise NotImplementedError("write your pallas kernel here")


# speedup vs baseline: 6.3296x; 6.3296x over previous
import jax
import jax.numpy as jnp
from jax.experimental import pallas as pl

B = 16
IMG = 224
PATCH = 16
FEAT = 768
HID = 768
EMB = 384
K = 2048
BETA = 0.25
EPS = 1e-8

N = 2 * B * 14 * 14
HALF = N // 2
T = 784
P = 392
NT = N // T
NP = HALF // P


def _leaky(x):
    return jnp.where(x >= 0, x, 0.1 * x)


def _dot(a, b):
    return jnp.dot(a, b, preferred_element_type=jnp.float32)


def _resblock(x, w1, b1, w2, b2):
    h = _leaky(_dot(x, w1) + b1)
    return x + (_dot(h, w2) + b2)


def _k1_kern(x_ref, pw, pb, e0w1, e0b1, e0w2, e0b2, e1w1, e1b1, e1w2, e1b2,
             vqw, dino_ref, feat_ref, f0_ref):
    x = x_ref[...]
    dino = _dot(x, pw[...]) + pb[...]
    t = _resblock(dino, e0w1[...], e0b1[...], e0w2[...], e0b2[...])
    feat = _resblock(t, e1w1[...], e1b1[...], e1w2[...], e1b2[...])
    dino_ref[...] = dino
    feat_ref[...] = feat
    f0_ref[...] = _dot(_leaky(feat), vqw[...])


def _vq_kern(za_ref, zb_ref, cbT_ref, cb_ref, c2_ref, qa_ref, qb_ref, jsd_ref, loss_ref):
    i = pl.program_id(0)
    cbT = cbT_ref[...]
    c2 = c2_ref[...]

    def half(z):
        z2 = jnp.sum(z * z, axis=1, keepdims=True)
        d = z2 + c2 - 2.0 * _dot(z, cbT)
        dmin = jnp.min(d, axis=1, keepdims=True)
        lanes = jax.lax.broadcasted_iota(jnp.int32, d.shape, 1)
        idx = jnp.min(jnp.where(d == dmin, lanes, K), axis=1)
        e = jnp.exp(dmin - d)
        p = e / jnp.sum(e, axis=1, keepdims=True)
        oh = (lanes == idx[:, None]).astype(jnp.float32)
        q = _dot(oh, cb_ref[...])
        return p, q

    za = za_ref[...]
    zb = zb_ref[...]
    pa, qa = half(za)
    pb, qb = half(zb)
    qa_ref[...] = qa
    qb_ref[...] = qb

    m = 0.5 * (pa + pb)
    lm = jnp.log(m + EPS)
    kl1 = jnp.sum(pa * (jnp.log(pa + EPS) - lm), axis=1)
    kl2 = jnp.sum(pb * (jnp.log(pb + EPS) - lm), axis=1)
    jsd_part = 0.5 * jnp.sum(kl1 + kl2)
    loss_part = jnp.sum((za - qa) ** 2) + jnp.sum((zb - qb) ** 2)

    @pl.when(i == 0)
    def _():
        jsd_ref[...] = jnp.zeros_like(jsd_ref)
        loss_ref[...] = jnp.zeros_like(loss_ref)

    jsd_ref[...] += jnp.reshape(jsd_part, (1, 1))
    loss_ref[...] += jnp.reshape(loss_part, (1, 1))


def _k3_kern(feat_ref, q0_ref, w, bias, vqw, f1_ref):
    cat = jnp.concatenate([feat_ref[...], q0_ref[...]], axis=1)
    feat2 = _dot(cat, w[...]) + bias[...]
    f1_ref[...] = _dot(_leaky(feat2), vqw[...])


def _k5_kern(q0_ref, q1_ref, dino_ref, w, bias,
             d0w1, d0b1, d0w2, d0b2, d1w1, d1b1, d1w2, d1b2,
             feat_ref, rec_ref):
    i = pl.program_id(0)
    cat = jnp.concatenate([q0_ref[...], q1_ref[...]], axis=1)
    f = _dot(cat, w[...]) + bias[...]
    r = _resblock(f, d0w1[...], d0b1[...], d0w2[...], d0b2[...])
    r = _resblock(r, d1w1[...], d1b1[...], d1w2[...], d1b2[...])
    feat_ref[...] = f
    part = jnp.sum((r - dino_ref[...]) ** 2)

    @pl.when(i == 0)
    def _():
        rec_ref[...] = jnp.zeros_like(rec_ref)

    rec_ref[...] += jnp.reshape(part, (1, 1))


def _full(shape):
    return pl.BlockSpec(shape, lambda i: (0, 0))


def _rows(bs, c):
    return pl.BlockSpec((bs, c), lambda i: (i, 0))


def _scalar():
    return pl.BlockSpec((1, 1), lambda i: (0, 0))


def _run_vq(f, cb):
    cbT = cb.T
    c2 = jnp.sum(cb * cb, axis=1).reshape(1, K)
    qa, qb, jsd_sum, loss_sum = pl.pallas_call(
        _vq_kern,
        grid=(NP,),
        in_specs=[
            pl.BlockSpec((P, EMB), lambda i: (i, 0)),
            pl.BlockSpec((P, EMB), lambda i: (i + NP, 0)),
            _full((EMB, K)),
            _full((K, EMB)),
            _full((1, K)),
        ],
        out_specs=[_rows(P, EMB), _rows(P, EMB), _scalar(), _scalar()],
        out_shape=[
            jax.ShapeDtypeStruct((HALF, EMB), jnp.float32),
            jax.ShapeDtypeStruct((HALF, EMB), jnp.float32),
            jax.ShapeDtypeStruct((1, 1), jnp.float32),
            jax.ShapeDtypeStruct((1, 1), jnp.float32),
        ],
    )(f, f, cbT, cb, c2)
    q = jnp.concatenate([qa, qb], axis=0)
    jsd = jsd_sum[0, 0] / HALF
    loss = (1.0 + BETA) * loss_sum[0, 0] / (N * EMB)
    return q, jsd, loss


def kernel(img, params):
    p = params
    ka, kb = jax.random.split(jax.random.key(1234))
    scale = jax.random.uniform(ka, (B, 3, 1, 1), jnp.float32, 0.9, 1.1)
    off = jax.random.uniform(kb, (B, 3, 1, 1), jnp.float32, -0.1, 0.1)
    x = jnp.concatenate([img, img * scale + off], axis=0)

    patches = (x.reshape(2 * B, 3, 14, PATCH, 14, PATCH)
                .transpose(0, 2, 4, 1, 3, 5)
                .reshape(N, 3 * PATCH * PATCH))

    pwT = p['pe_w'].reshape(FEAT, 3 * PATCH * PATCH).T
    row = lambda b: b.reshape(1, -1)

    dino, feat, f0 = pl.pallas_call(
        _k1_kern,
        grid=(NT,),
        in_specs=[
            _rows(T, 768),
            _full((768, FEAT)), _full((1, FEAT)),
            _full((HID, HID)), _full((1, HID)), _full((HID, HID)), _full((1, HID)),
            _full((HID, HID)), _full((1, HID)), _full((HID, HID)), _full((1, HID)),
            _full((HID, EMB)),
        ],
        out_specs=[_rows(T, FEAT), _rows(T, HID), _rows(T, EMB)],
        out_shape=[
            jax.ShapeDtypeStruct((N, FEAT), jnp.float32),
            jax.ShapeDtypeStruct((N, HID), jnp.float32),
            jax.ShapeDtypeStruct((N, EMB), jnp.float32),
        ],
    )(patches, pwT, row(p['pe_b']),
      p['enc0_w1'].T, row(p['enc0_b1']), p['enc0_w2'].T, row(p['enc0_b2']),
      p['enc1_w1'].T, row(p['enc1_b1']), p['enc1_w2'].T, row(p['enc1_b2']),
      p['vq0_in_w'].T)

    q0, jsd0, l0 = _run_vq(f0, p['cb0'])

    voT = p['vqout0_w'].T
    f1 = pl.pallas_call(
        _k3_kern,
        grid=(NT,),
        in_specs=[
            _rows(T, HID), _rows(T, EMB),
            _full((HID + EMB, HID)), _full((1, HID)),
            _full((HID, EMB)),
        ],
        out_specs=_rows(T, EMB),
        out_shape=jax.ShapeDtypeStruct((N, EMB), jnp.float32),
    )(feat, q0, voT, row(p['vqout0_b']), p['vq1_in_w'].T)

    q1, jsd1, l1 = _run_vq(f1, p['cb1'])

    agT = p['agg_w'].T
    feat3, rec_sum = pl.pallas_call(
        _k5_kern,
        grid=(NT,),
        in_specs=[
            _rows(T, EMB), _rows(T, EMB), _rows(T, FEAT),
            _full((2 * EMB, HID)), _full((1, HID)),
            _full((HID, HID)), _full((1, HID)), _full((HID, HID)), _full((1, HID)),
            _full((HID, HID)), _full((1, HID)), _full((HID, HID)), _full((1, HID)),
        ],
        out_specs=[_rows(T, HID), _scalar()],
        out_shape=[
            jax.ShapeDtypeStruct((N, HID), jnp.float32),
            jax.ShapeDtypeStruct((1, 1), jnp.float32),
        ],
    )(q0, q1, dino, agT, row(p['agg_b']),
      p['dec0_w1'].T, row(p['dec0_b1']), p['dec0_w2'].T, row(p['dec0_b2']),
      p['dec1_w1'].T, row(p['dec1_b1']), p['dec1_w2'].T, row(p['dec1_b2']))

    untok = lambda a, c: a[:HALF].reshape(B, 14, 14, c).transpose(0, 3, 1, 2)
    recon_loss = rec_sum[0, 0] / (N * FEAT)
    contra_loss = jsd0 - 0.1 * jsd1
    return (untok(feat3, HID), untok(q0, EMB), untok(q1, EMB),
            recon_loss, contra_loss, l0 + l1)
